# baseline (speedup 1.0000x reference)
import functools

import jax
import jax.numpy as jnp
from jax import lax
from jax.experimental import pallas as pl
from jax.experimental.pallas import tpu as pltpu
from jax.experimental.pallas import tpu_sc as plsc

N_REAL = 10000
N_PAD = 10240
E_TOT = 320000
F = 128
G_GRAPHS = 128
C_OUT = 2

NC, NS = 2, 16
NW = NC * NS
CHUNK = 80
NCHUNK = 125
EDGES_PER_W = NCHUNK * CHUNK
ROWS_PER_TILE = N_PAD // NS
BLK = 1024
NB = N_PAD // BLK

@functools.cache
def _make_sc_degree():
    mesh = plsc.VectorSubcoreMesh(core_axis_name="c", subcore_axis_name="s",
                                  num_cores=NC, num_subcores=NS)
    return pl.kernel(
        _sc_degree_body,
        out_type=jax.ShapeDtypeStruct((NC, N_PAD), jnp.float32),
        mesh=mesh,
        scratch_types=[
            pltpu.VMEM((CHUNK,), jnp.int32),
            pltpu.VMEM((CHUNK,), jnp.int32),
            pltpu.VMEM((CHUNK,), jnp.int32),
            pltpu.VMEM((CHUNK,), jnp.int32),
            pltpu.VMEM((CHUNK,), jnp.float32),
            pltpu.VMEM((ROWS_PER_TILE,), jnp.float32),
            pltpu.VMEM_SHARED((N_PAD,), jnp.float32),
            pltpu.SemaphoreType.DMA,
            pltpu.SemaphoreType.DMA,
            pltpu.SemaphoreType.DMA,
            pltpu.SemaphoreType.DMA,
            pltpu.SemaphoreType.DMA,
            pltpu.SemaphoreType.DMA,
        ],
    )


def _sc_degree_body(cols_hbm, deg_out, col0, col1, col2, col3,
                    ones_v, zbuf, deg_sh, semi0, semi1, semi2, semi3,
                    sems0, sems1):
    c = lax.axis_index("c")
    s = lax.axis_index("s")
    w = s * NC + c
    cols = (col0, col1, col2, col3)
    semi = (semi0, semi1, semi2, semi3)
    sems = (sems0, sems1)
    sl = pl.ds(s * ROWS_PER_TILE, ROWS_PER_TILE)

    def zb(t, _):
        zbuf[pl.ds(t * 16, 16)] = jnp.zeros((16,), jnp.float32)
        return 0

    lax.fori_loop(0, ROWS_PER_TILE // 16, zb, 0)
    pltpu.sync_copy(zbuf, deg_sh.at[sl])
    for t in range(CHUNK // 16):
        ones_v[pl.ds(t * 16, 16)] = jnp.ones((16,), jnp.float32)
    plsc.subcore_barrier()

    def cidx_src(j):
        return cols_hbm.at[pl.ds(w * EDGES_PER_W + j * CHUNK, CHUNK)]

    def issue_idx(j, k):
        pltpu.async_copy(cidx_src(j), cols[k], semi[k])

    def wait_idx(j, k):
        pltpu.make_async_copy(cidx_src(j), cols[k], semi[k]).wait()

    def issue_scat(k, m):
        pltpu.async_copy(ones_v, deg_sh.at[cols[k]], sems[m], add=True)

    def wait_scat(m):
        pltpu.make_async_copy(ones_v, deg_sh.at[col0], sems[m]).wait()

    for j in range(4):
        issue_idx(j, j)
    wait_idx(0, 0)
    issue_scat(0, 0)
    wait_idx(1, 1)
    issue_scat(1, 1)

    def body(t, _):
        for k in range(4):
            j = 4 * t + 2 + k
            jb = (2 + k) % 4
            jm = k % 2
            wait_scat(jm)
            issue_idx(j + 2, (jb + 2) % 4)
            wait_idx(j, jb)
            issue_scat(jb, jm)
        return 0

    lax.fori_loop(0, 30, body, 0)
    for j in (122, 123, 124):
        jb = j % 4
        jm = j % 2
        wait_scat(jm)
        if j == 122:
            issue_idx(124, 0)
        wait_idx(j, jb)
        issue_scat(jb, jm)
    wait_scat((123 - 2) % 2)
    wait_scat((124 - 2) % 2)
    plsc.subcore_barrier()
    pltpu.sync_copy(deg_sh.at[sl], deg_out.at[c, sl])


@functools.cache
def _make_sc_scatter():
    mesh = plsc.VectorSubcoreMesh(core_axis_name="c", subcore_axis_name="s",
                                  num_cores=NC, num_subcores=NS)
    return pl.kernel(
        _sc_scatter_body,
        out_type=jax.ShapeDtypeStruct((NC, N_PAD, F), jnp.float32),
        mesh=mesh,
        scratch_types=[
            [pltpu.VMEM((CHUNK,), jnp.int32)] * 4,
            [pltpu.VMEM((CHUNK,), jnp.int32)] * 4,
            [pltpu.VMEM((CHUNK, F), jnp.float32)] * 4,
            pltpu.VMEM_SHARED((N_PAD, F), jnp.float32),
            [pltpu.SemaphoreType.DMA] * 4,
            [pltpu.SemaphoreType.DMA] * 2,
            [pltpu.SemaphoreType.DMA] * 4,
            [pltpu.SemaphoreType.DMA] * 4,
        ],
    )


def _sc_scatter_body(rows_hbm, cols_hbm, y_hbm, acc_out,
                     rbuf, cbuf, gbuf, acc_sh, semg, sems, semr, semc):
    c = lax.axis_index("c")
    s = lax.axis_index("s")
    w = s * NC + c
    sl = pl.ds(s * ROWS_PER_TILE, ROWS_PER_TILE)

    @pl.when(c == 0)
    def _():
        pltpu.sync_copy(y_hbm.at[sl], acc_sh.at[sl])

    @pl.when(c != 0)
    def _():
        def zrow(r, _):
            for t in range(F // 16):
                gbuf[0][r, pl.ds(t * 16, 16)] = jnp.zeros((16,), jnp.float32)
            return 0

        lax.fori_loop(0, CHUNK, zrow, 0)
        for q in range(ROWS_PER_TILE // CHUNK):
            pltpu.sync_copy(
                gbuf[0],
                acc_sh.at[pl.ds(s * ROWS_PER_TILE + q * CHUNK, CHUNK)])

    plsc.subcore_barrier()

    def rsrc(j):
        return rows_hbm.at[pl.ds(w * EDGES_PER_W + j * CHUNK, CHUNK)]

    def csrc(j):
        return cols_hbm.at[pl.ds(w * EDGES_PER_W + j * CHUNK, CHUNK)]

    def i_r(j, k):
        pltpu.async_copy(rsrc(j), rbuf[k], semr[k])

    def w_r(j, k):
        pltpu.make_async_copy(rsrc(j), rbuf[k], semr[k]).wait()

    def i_c(j, k):
        pltpu.async_copy(csrc(j), cbuf[k], semc[k])

    def w_c(j, k):
        pltpu.make_async_copy(csrc(j), cbuf[k], semc[k]).wait()

    def i_g(k):
        pltpu.async_copy(y_hbm.at[rbuf[k]], gbuf[k], semg[k])

    def w_g(k):
        pltpu.make_async_copy(y_hbm.at[rbuf[k]], gbuf[k], semg[k]).wait()

    def i_s(k, m):
        pltpu.async_copy(gbuf[k], acc_sh.at[cbuf[k]], sems[m], add=True)

    def w_s(m):
        pltpu.make_async_copy(gbuf[0], acc_sh.at[cbuf[0]], sems[m]).wait()

    def pos(j, jb, jm, first=False, lastj=NCHUNK - 1):
        w_g(jb)
        if not first:
            w_s(jm)
        w_c(j, jb)
        i_s(jb, jm)
        nxt = (jb + 2) % 4
        if not isinstance(j, int) or j + 2 <= lastj:
            i_c(j + 2, nxt)
            w_r(j + 2, nxt)
            i_g(nxt)
        if not isinstance(j, int) or j + 4 <= lastj:
            i_r(j + 4, jb)

    for k in range(4):
        i_r(k, k)
    i_c(0, 0)
    i_c(1, 1)
    w_r(0, 0)
    i_g(0)
    w_r(1, 1)
    i_g(1)
    pos(0, 0, 0, first=True)
    pos(1, 1, 1, first=True)

    def body(t, _):
        for k in range(4):
            pos(4 * t + 2 + k, (2 + k) % 4, k % 2)
        return 0

    lax.fori_loop(0, (NCHUNK - 9) // 4, body, 0)
    for j in range(NCHUNK - 7, NCHUNK):
        pos(j, j % 4, j % 2)
    w_s((NCHUNK - 2) % 2)
    w_s((NCHUNK - 1) % 2)
    plsc.subcore_barrier()
    pltpu.sync_copy(acc_sh.at[sl], acc_out.at[c, sl])


def _tc_y_body(x_ref, w1_ref, deg_ref, y_ref):
    d = deg_ref[0, :] + deg_ref[1, :] + 1.0
    dis = lax.rsqrt(d)
    xw = jnp.dot(x_ref[...], w1_ref[...], preferred_element_type=jnp.float32)
    y_ref[...] = xw * dis[:, None]


def _tc_y(x_pad, w1, deg_part):
    return pl.pallas_call(
        _tc_y_body,
        grid=(NB,),
        in_specs=[
            pl.BlockSpec((BLK, F), lambda i: (i, 0)),
            pl.BlockSpec((F, F), lambda i: (0, 0)),
            pl.BlockSpec((NC, BLK), lambda i: (0, i)),
        ],
        out_specs=pl.BlockSpec((BLK, F), lambda i: (i, 0)),
        out_shape=jax.ShapeDtypeStruct((N_PAD, F), jnp.float32),
    )(x_pad, w1, deg_part)


def _tc_fin_body(acc_ref, deg_ref, batch_ref, b1_ref, w2_ref, b2_ref,
                 out_ref, pooled_scr):
    i = pl.program_id(0)

    @pl.when(i == 0)
    def _():
        pooled_scr[...] = jnp.full((G_GRAPHS, F), -jnp.inf, jnp.float32)

    d = deg_ref[0, :] + deg_ref[1, :] + 1.0
    dis = lax.rsqrt(d)
    conv = (acc_ref[0] + acc_ref[1]) * dis[:, None] + b1_ref[...]
    h = jnp.maximum(conv, 0.0)
    rid = i * BLK + lax.broadcasted_iota(jnp.int32, (BLK, 1), 0)
    hp = jnp.where(rid < N_REAL, h, -jnp.inf)
    bvec = batch_ref[...]
    glo = jnp.min(bvec)
    ghi = jnp.max(bvec)

    def gbody(g, _):
        m = bvec == g
        colmax = jnp.max(jnp.where(m, hp, -jnp.inf), axis=0, keepdims=True)
        cur = pooled_scr[pl.ds(g, 1), :]
        pooled_scr[pl.ds(g, 1), :] = jnp.maximum(cur, colmax)
        return 0

    lax.fori_loop(glo, ghi + 1, gbody, 0)

    @pl.when(i == NB - 1)
    def _():
        pooled = pooled_scr[...]
        logits = jnp.dot(pooled, w2_ref[...],
                         preferred_element_type=jnp.float32) + b2_ref[...]
        mx = jnp.max(logits, axis=-1, keepdims=True)
        lse = mx + jnp.log(jnp.sum(jnp.exp(logits - mx), axis=-1, keepdims=True))
        out_ref[...] = logits - lse


def _tc_fin(acc_part, deg_part, batch_pad, b1, w2, b2):
    return pl.pallas_call(
        _tc_fin_body,
        grid=(NB,),
        in_specs=[
            pl.BlockSpec((NC, BLK, F), lambda i: (0, i, 0)),
            pl.BlockSpec((NC, BLK), lambda i: (0, i)),
            pl.BlockSpec((BLK, 1), lambda i: (i, 0)),
            pl.BlockSpec((1, F), lambda i: (0, 0)),
            pl.BlockSpec((F, C_OUT), lambda i: (0, 0)),
            pl.BlockSpec((1, C_OUT), lambda i: (0, 0)),
        ],
        out_specs=pl.BlockSpec((G_GRAPHS, C_OUT), lambda i: (0, 0)),
        out_shape=jax.ShapeDtypeStruct((G_GRAPHS, C_OUT), jnp.float32),
        scratch_shapes=[pltpu.VMEM((G_GRAPHS, F), jnp.float32)],
        compiler_params=pltpu.CompilerParams(
            dimension_semantics=("arbitrary",)),
    )(acc_part, deg_part, batch_pad, b1, w2, b2)


def kernel(x, edge_index, batch, W1, b1, W2, b2):
    x_pad = jnp.pad(x, ((0, N_PAD - N_REAL), (0, 0)))
    batch_pad = jnp.pad(batch, (0, N_PAD - N_REAL),
                        constant_values=G_GRAPHS - 1)
    e_per_w = E_TOT // NW
    pad_w = EDGES_PER_W - e_per_w
    if pad_w:
        dummy = jnp.full((NW, pad_w), N_PAD - 1, jnp.int32)
        rows_flat = jnp.concatenate(
            [edge_index[0].reshape(NW, e_per_w), dummy], axis=1).reshape(-1)
        cols_flat = jnp.concatenate(
            [edge_index[1].reshape(NW, e_per_w), dummy], axis=1).reshape(-1)
    else:
        rows_flat = edge_index[0]
        cols_flat = edge_index[1]

    deg_part = _make_sc_degree()(cols_flat)
    y = _tc_y(x_pad, W1, deg_part)
    acc_part = _make_sc_scatter()(rows_flat, cols_flat, y)
    out = _tc_fin(acc_part, deg_part, batch_pad.reshape(-1, 1),
                  b1.reshape(1, F), W2, b2.reshape(1, C_OUT))
    return out


# speedup vs baseline: 1.0153x; 1.0153x over previous
import functools

import jax
import jax.numpy as jnp
from jax import lax
from jax.experimental import pallas as pl
from jax.experimental.pallas import tpu as pltpu
from jax.experimental.pallas import tpu_sc as plsc

N_REAL = 10000
N_PAD = 10240
E_TOT = 320000
F = 128
G_GRAPHS = 128
C_OUT = 2

NC, NS = 2, 16
NW = NC * NS
CHUNK = 80
NCHUNK = 125
EDGES_PER_W = NCHUNK * CHUNK
ROWS_PER_TILE = N_PAD // NS
BLK = 512
NB = N_PAD // BLK

@functools.cache
def _make_sc_degree():
    mesh = plsc.VectorSubcoreMesh(core_axis_name="c", subcore_axis_name="s",
                                  num_cores=NC, num_subcores=NS)
    return pl.kernel(
        _sc_degree_body,
        out_type=jax.ShapeDtypeStruct((NC, N_PAD), jnp.float32),
        mesh=mesh,
        scratch_types=[
            pltpu.VMEM((CHUNK,), jnp.int32),
            pltpu.VMEM((CHUNK,), jnp.int32),
            pltpu.VMEM((CHUNK,), jnp.int32),
            pltpu.VMEM((CHUNK,), jnp.int32),
            pltpu.VMEM((CHUNK,), jnp.float32),
            pltpu.VMEM((ROWS_PER_TILE,), jnp.float32),
            pltpu.VMEM_SHARED((N_PAD,), jnp.float32),
            pltpu.SemaphoreType.DMA,
            pltpu.SemaphoreType.DMA,
            pltpu.SemaphoreType.DMA,
            pltpu.SemaphoreType.DMA,
            pltpu.SemaphoreType.DMA,
            pltpu.SemaphoreType.DMA,
        ],
    )


def _sc_degree_body(cols_hbm, deg_out, col0, col1, col2, col3,
                    ones_v, zbuf, deg_sh, semi0, semi1, semi2, semi3,
                    sems0, sems1):
    c = lax.axis_index("c")
    s = lax.axis_index("s")
    w = s * NC + c
    cols = (col0, col1, col2, col3)
    semi = (semi0, semi1, semi2, semi3)
    sems = (sems0, sems1)
    sl = pl.ds(s * ROWS_PER_TILE, ROWS_PER_TILE)

    def cidx_src(j):
        return cols_hbm.at[pl.ds(w * EDGES_PER_W + j * CHUNK, CHUNK)]

    def issue_idx(j, k):
        pltpu.async_copy(cidx_src(j), cols[k], semi[k])

    def wait_idx(j, k):
        pltpu.make_async_copy(cidx_src(j), cols[k], semi[k]).wait()

    def issue_scat(k, m):
        pltpu.async_copy(ones_v, deg_sh.at[cols[k]], sems[m], add=True)

    def wait_scat(m):
        pltpu.make_async_copy(ones_v, deg_sh.at[col0], sems[m]).wait()

    for j in range(4):
        issue_idx(j, j)

    def zb(t, _):
        zbuf[pl.ds(t * 16, 16)] = jnp.zeros((16,), jnp.float32)
        return 0

    lax.fori_loop(0, ROWS_PER_TILE // 16, zb, 0)
    pltpu.sync_copy(zbuf, deg_sh.at[sl])
    for t in range(CHUNK // 16):
        ones_v[pl.ds(t * 16, 16)] = jnp.ones((16,), jnp.float32)
    plsc.subcore_barrier()

    wait_idx(0, 0)
    issue_scat(0, 0)
    wait_idx(1, 1)
    issue_scat(1, 1)

    def body(t, _):
        for k in range(4):
            j = 4 * t + 2 + k
            jb = (2 + k) % 4
            jm = k % 2
            wait_scat(jm)
            issue_idx(j + 2, (jb + 2) % 4)
            wait_idx(j, jb)
            issue_scat(jb, jm)
        return 0

    lax.fori_loop(0, 30, body, 0)
    for j in (122, 123, 124):
        jb = j % 4
        jm = j % 2
        wait_scat(jm)
        if j == 122:
            issue_idx(124, 0)
        wait_idx(j, jb)
        issue_scat(jb, jm)
    wait_scat((123 - 2) % 2)
    wait_scat((124 - 2) % 2)
    plsc.subcore_barrier()
    pltpu.sync_copy(deg_sh.at[sl], deg_out.at[c, sl])


@functools.cache
def _make_sc_scatter():
    mesh = plsc.VectorSubcoreMesh(core_axis_name="c", subcore_axis_name="s",
                                  num_cores=NC, num_subcores=NS)
    return pl.kernel(
        _sc_scatter_body,
        out_type=jax.ShapeDtypeStruct((NC, N_PAD, F), jnp.float32),
        mesh=mesh,
        scratch_types=[
            [pltpu.VMEM((CHUNK,), jnp.int32)] * 4,
            [pltpu.VMEM((CHUNK,), jnp.int32)] * 4,
            [pltpu.VMEM((CHUNK, F), jnp.float32)] * 4,
            pltpu.VMEM_SHARED((N_PAD, F), jnp.float32),
            [pltpu.SemaphoreType.DMA] * 4,
            [pltpu.SemaphoreType.DMA] * 2,
            [pltpu.SemaphoreType.DMA] * 4,
            [pltpu.SemaphoreType.DMA] * 4,
        ],
    )


def _sc_scatter_body(rows_hbm, cols_hbm, y_hbm, acc_out,
                     rbuf, cbuf, gbuf, acc_sh, semg, sems, semr, semc):
    c = lax.axis_index("c")
    s = lax.axis_index("s")
    w = s * NC + c
    sl = pl.ds(s * ROWS_PER_TILE, ROWS_PER_TILE)

    def rsrc(j):
        return rows_hbm.at[pl.ds(w * EDGES_PER_W + j * CHUNK, CHUNK)]

    def csrc(j):
        return cols_hbm.at[pl.ds(w * EDGES_PER_W + j * CHUNK, CHUNK)]

    def i_r(j, k):
        pltpu.async_copy(rsrc(j), rbuf[k], semr[k])

    def w_r(j, k):
        pltpu.make_async_copy(rsrc(j), rbuf[k], semr[k]).wait()

    def i_c(j, k):
        pltpu.async_copy(csrc(j), cbuf[k], semc[k])

    def w_c(j, k):
        pltpu.make_async_copy(csrc(j), cbuf[k], semc[k]).wait()

    def i_g(k):
        pltpu.async_copy(y_hbm.at[rbuf[k]], gbuf[k], semg[k])

    def w_g(k):
        pltpu.make_async_copy(y_hbm.at[rbuf[k]], gbuf[k], semg[k]).wait()

    def i_s(k, m):
        pltpu.async_copy(gbuf[k], acc_sh.at[cbuf[k]], sems[m], add=True)

    def w_s(m):
        pltpu.make_async_copy(gbuf[0], acc_sh.at[cbuf[0]], sems[m]).wait()

    def pos(j, jb, jm, first=False, lastj=NCHUNK - 1):
        w_g(jb)
        if not first:
            w_s(jm)
        w_c(j, jb)
        i_s(jb, jm)
        nxt = (jb + 2) % 4
        if not isinstance(j, int) or j + 2 <= lastj:
            i_c(j + 2, nxt)
            w_r(j + 2, nxt)
            i_g(nxt)
        if not isinstance(j, int) or j + 4 <= lastj:
            i_r(j + 4, jb)

    for k in range(4):
        i_r(k, k)
    i_c(0, 0)
    i_c(1, 1)
    w_r(0, 0)
    i_g(0)
    w_r(1, 1)
    i_g(1)

    @pl.when(c == 0)
    def _():
        pltpu.sync_copy(y_hbm.at[sl], acc_sh.at[sl])

    @pl.when(c != 0)
    def _():
        def zrow(r, _):
            for t in range(F // 16):
                gbuf[3][r, pl.ds(t * 16, 16)] = jnp.zeros((16,), jnp.float32)
            return 0

        lax.fori_loop(0, CHUNK, zrow, 0)
        for q in range(ROWS_PER_TILE // CHUNK):
            pltpu.sync_copy(
                gbuf[3],
                acc_sh.at[pl.ds(s * ROWS_PER_TILE + q * CHUNK, CHUNK)])

    plsc.subcore_barrier()
    pos(0, 0, 0, first=True)
    pos(1, 1, 1, first=True)

    def body(t, _):
        for k in range(4):
            pos(4 * t + 2 + k, (2 + k) % 4, k % 2)
        return 0

    lax.fori_loop(0, (NCHUNK - 9) // 4, body, 0)
    for j in range(NCHUNK - 7, NCHUNK):
        pos(j, j % 4, j % 2)
    w_s((NCHUNK - 2) % 2)
    w_s((NCHUNK - 1) % 2)
    plsc.subcore_barrier()
    pltpu.sync_copy(acc_sh.at[sl], acc_out.at[c, sl])


def _tc_y_body(x_ref, w1_ref, deg_ref, y_ref):
    d = deg_ref[0, :] + deg_ref[1, :] + 1.0
    dis = lax.rsqrt(d)
    xw = jnp.dot(x_ref[...], w1_ref[...], preferred_element_type=jnp.float32)
    y_ref[...] = xw * dis[:, None]


def _tc_y(x_pad, w1, deg_part):
    return pl.pallas_call(
        _tc_y_body,
        grid=(NB,),
        in_specs=[
            pl.BlockSpec((BLK, F), lambda i: (i, 0)),
            pl.BlockSpec((F, F), lambda i: (0, 0)),
            pl.BlockSpec((NC, BLK), lambda i: (0, i)),
        ],
        out_specs=pl.BlockSpec((BLK, F), lambda i: (i, 0)),
        out_shape=jax.ShapeDtypeStruct((N_PAD, F), jnp.float32),
    )(x_pad, w1, deg_part)


def _tc_fin_body(acc_ref, deg_ref, batch_ref, b1_ref, w2_ref, b2_ref,
                 out_ref, pooled_scr):
    i = pl.program_id(0)

    @pl.when(i == 0)
    def _():
        pooled_scr[...] = jnp.full((G_GRAPHS, F), -jnp.inf, jnp.float32)

    d = deg_ref[0, :] + deg_ref[1, :] + 1.0
    dis = lax.rsqrt(d)
    conv = (acc_ref[0] + acc_ref[1]) * dis[:, None] + b1_ref[...]
    h = jnp.maximum(conv, 0.0)
    rid = i * BLK + lax.broadcasted_iota(jnp.int32, (BLK, 1), 0)
    hp = jnp.where(rid < N_REAL, h, -jnp.inf)
    bvec = batch_ref[...]
    glo = jnp.min(bvec)
    ghi = jnp.max(bvec)

    def gbody(g, _):
        m = bvec == g
        colmax = jnp.max(jnp.where(m, hp, -jnp.inf), axis=0, keepdims=True)
        cur = pooled_scr[pl.ds(g, 1), :]
        pooled_scr[pl.ds(g, 1), :] = jnp.maximum(cur, colmax)
        return 0

    lax.fori_loop(glo, ghi + 1, gbody, 0)

    @pl.when(i == NB - 1)
    def _():
        pooled = pooled_scr[...]
        logits = jnp.dot(pooled, w2_ref[...],
                         preferred_element_type=jnp.float32) + b2_ref[...]
        mx = jnp.max(logits, axis=-1, keepdims=True)
        lse = mx + jnp.log(jnp.sum(jnp.exp(logits - mx), axis=-1, keepdims=True))
        out_ref[...] = logits - lse


def _tc_fin(acc_part, deg_part, batch_pad, b1, w2, b2):
    return pl.pallas_call(
        _tc_fin_body,
        grid=(NB,),
        in_specs=[
            pl.BlockSpec((NC, BLK, F), lambda i: (0, i, 0)),
            pl.BlockSpec((NC, BLK), lambda i: (0, i)),
            pl.BlockSpec((BLK, 1), lambda i: (i, 0)),
            pl.BlockSpec((1, F), lambda i: (0, 0)),
            pl.BlockSpec((F, C_OUT), lambda i: (0, 0)),
            pl.BlockSpec((1, C_OUT), lambda i: (0, 0)),
        ],
        out_specs=pl.BlockSpec((G_GRAPHS, C_OUT), lambda i: (0, 0)),
        out_shape=jax.ShapeDtypeStruct((G_GRAPHS, C_OUT), jnp.float32),
        scratch_shapes=[pltpu.VMEM((G_GRAPHS, F), jnp.float32)],
        compiler_params=pltpu.CompilerParams(
            dimension_semantics=("arbitrary",)),
    )(acc_part, deg_part, batch_pad, b1, w2, b2)


def kernel(x, edge_index, batch, W1, b1, W2, b2):
    x_pad = jnp.pad(x, ((0, N_PAD - N_REAL), (0, 0)))
    batch_pad = jnp.pad(batch, (0, N_PAD - N_REAL),
                        constant_values=G_GRAPHS - 1)
    e_per_w = E_TOT // NW
    pad_w = EDGES_PER_W - e_per_w
    if pad_w:
        dummy = jnp.full((NW, pad_w), N_PAD - 1, jnp.int32)
        rows_flat = jnp.concatenate(
            [edge_index[0].reshape(NW, e_per_w), dummy], axis=1).reshape(-1)
        cols_flat = jnp.concatenate(
            [edge_index[1].reshape(NW, e_per_w), dummy], axis=1).reshape(-1)
    else:
        rows_flat = edge_index[0]
        cols_flat = edge_index[1]

    deg_part = _make_sc_degree()(cols_flat)
    y = _tc_y(x_pad, W1, deg_part)
    acc_part = _make_sc_scatter()(rows_flat, cols_flat, y)
    out = _tc_fin(acc_part, deg_part, batch_pad.reshape(-1, 1),
                  b1.reshape(1, F), W2, b2.reshape(1, C_OUT))
    return out
